# baseline (speedup 1.0000x reference)
import jax
import jax.numpy as jnp
from jax import lax
from jax.experimental import pallas as pl

_NUM_PIECES = 32
_HW = 64
_C = 128


def _extract_block(ids_ref, board_ref, out_ref):
    ids = ids_ref[...]
    bb = ids.shape[0]

    t = lax.broadcasted_iota(jnp.int32, (bb, _NUM_PIECES, _HW), 1)
    maskf = (ids[:, None, :] == t + 1).astype(jnp.float32)

    h_i = lax.broadcasted_iota(jnp.int32, (_HW, _HW), 0)
    k_i = lax.broadcasted_iota(jnp.int32, (_HW, _HW), 1)
    lt = (h_i < k_i).astype(jnp.float32)
    earlier = lax.dot_general(
        maskf, lt, dimension_numbers=(((2,), (0,)), ((), ())),
        preferred_element_type=jnp.float32)
    onehot = maskf * (earlier < 0.5).astype(jnp.float32)

    z = jnp.zeros_like(onehot)
    sel = jnp.concatenate(
        [jnp.concatenate([onehot, z], axis=2),
         jnp.concatenate([z, onehot], axis=2)], axis=1)

    v = board_ref[...]
    o = lax.dot_general(
        sel, v, dimension_numbers=(((2,), (2,)), ((0,), (0,))),
        preferred_element_type=jnp.float32)

    oc = jnp.concatenate([o[:, :_NUM_PIECES, :],
                          o[:, _NUM_PIECES:, :]], axis=2)

    l_i = lax.broadcasted_iota(jnp.int32, (_C, _C), 0)
    c_i = lax.broadcasted_iota(jnp.int32, (_C, _C), 1)
    target_c = jnp.where(l_i < _HW, 2 * l_i, 2 * (l_i - _HW) + 1)
    perm = (c_i == target_c).astype(jnp.float32)
    out_ref[...] = lax.dot_general(
        oc, perm, dimension_numbers=(((2,), (0,)), ((), ())),
        preferred_element_type=jnp.float32)


def kernel(full_board_vector, piece_ids):
    B, C, H, W = full_board_vector.shape
    HW = H * W
    flat_ids = piece_ids.reshape(B, HW)
    board_rows = full_board_vector.reshape(B, C * HW // _C, _C)

    BB = 128
    grid = (B // BB,)
    return pl.pallas_call(
        _extract_block,
        grid=grid,
        in_specs=[
            pl.BlockSpec((BB, HW), lambda i: (i, 0)),
            pl.BlockSpec((BB, C * HW // _C, _C), lambda i: (i, 0, 0)),
        ],
        out_specs=pl.BlockSpec((BB, _NUM_PIECES, C), lambda i: (i, 0, 0)),
        out_shape=jax.ShapeDtypeStruct((B, _NUM_PIECES, C), jnp.float32),
    )(flat_ids, board_rows)
